# baseline (speedup 1.0000x reference)
import jax
import jax.numpy as jnp
from jax import lax
from jax.experimental import pallas as pl
from jax.experimental.pallas import tpu as pltpu
from jax.experimental.pallas import tpu_sc as plsc

_NXM, _NYM = 240, 64
_G = 8
_COLS = _G * _NYM
_NP = 480
_IN, _H1, _HID = 768, 512, 128

_PROWS, _PCOLS = 738, 528
_FLAT = _PROWS * _PCOLS
_TROWS = 23
_NTILES = 32
_WIN = (_TROWS + 2) * _PCOLS
_OUT = _TROWS * _PCOLS


def _field_body(pv_ref, w1_ref, b1_ref, w2_ref, b2_ref, wg_ref, t_ref):
    f32 = jnp.float32
    a = jnp.dot(pv_ref[...], w1_ref[...], preferred_element_type=f32)
    a = a + b1_ref[...]
    a = jnp.maximum(a, 0.0) + jnp.log1p(jnp.exp(-jnp.abs(a)))
    h = jnp.dot(a, w2_ref[...], preferred_element_type=f32) + b2_ref[...]
    p0 = jnp.dot(h, wg_ref[0:_HID, :], preferred_element_type=f32)

    r4 = lax.broadcasted_iota(jnp.int32, (_NP, _COLS), 0)
    c4 = lax.broadcasted_iota(jnp.int32, (_NP, _COLS), 1)
    mask = ((r4 // 60 == c4 // 64) & (r4 % 4 == (c4 % 64) // 16)).astype(f32)
    xu = lax.broadcasted_iota(jnp.int32, (_NXM, _NP), 0)
    ru = lax.broadcasted_iota(jnp.int32, (_NXM, _NP), 1)
    u = ((ru % 60) // 4 == xu // 16).astype(f32)

    r_x = lax.broadcasted_iota(jnp.int32, (_NXM, _COLS), 0)
    c_i = lax.broadcasted_iota(jnp.int32, (_NXM, _COLS), 1)
    y = c_i % _NYM
    deg = (1
           + (r_x > 0).astype(jnp.int32) + (r_x < _NXM - 1).astype(jnp.int32)
           + (y > 0).astype(jnp.int32) + (y < _NYM - 1).astype(jnp.int32))
    rs = lax.rsqrt(deg.astype(f32))
    xp_f = (r_x // 16).astype(f32)
    yp_f = (y // 16).astype(f32)
    xi_f = (r_x % 16).astype(f32) * (1.0 / 15.0)
    yi_f = (y % 16).astype(f32) * (1.0 / 15.0)

    t_ref[...] = jnp.zeros((_PROWS, _PCOLS), f32)
    for c in range(3):
        z = p0[:, c:c + 1] * mask
        b = jnp.dot(u, z, preferred_element_type=f32)
        hx = (b
              + xp_f * wg_ref[_HID + 0:_HID + 1, c:c + 1]
              + yp_f * wg_ref[_HID + 1:_HID + 2, c:c + 1]
              + xi_f * wg_ref[_HID + 2:_HID + 3, c:c + 1]
              + yi_f * wg_ref[_HID + 3:_HID + 4, c:c + 1])
        t = rs * hx
        for g in range(_G):
            t_ref[c * 242 + 2:c * 242 + 242, g * 66 + 1:g * 66 + 65] = (
                t[:, g * _NYM:(g + 1) * _NYM])


def _sc_body(t_hbm, o_hbm, win, obuf):
    wid = lax.axis_index("s") * 2 + lax.axis_index("c")
    base = wid * _OUT
    pltpu.sync_copy(t_hbm.at[pl.ds(base, _WIN)], win)

    def body(i, carry):
        off = i * 16
        mid = win[pl.ds(off + _PCOLS, 16)]
        up = win[pl.ds(off, 16)]
        dn = win[pl.ds(off + 2 * _PCOLS, 16)]
        lf = win[pl.ds(off + _PCOLS - 1, 16)]
        rt = win[pl.ds(off + _PCOLS + 1, 16)]
        obuf[pl.ds(off, 16)] = mid + up + dn + lf + rt
        return carry

    lax.fori_loop(0, _OUT // 16, body, 0)
    pltpu.sync_copy(obuf, o_hbm.at[pl.ds(base + _PCOLS, _OUT)])


def _final_body(o_ref, bg_ref, out_ref):
    f32 = jnp.float32
    r_x = lax.broadcasted_iota(jnp.int32, (_NXM, _NYM), 0)
    y = lax.broadcasted_iota(jnp.int32, (_NXM, _NYM), 1)
    deg = (1
           + (r_x > 0).astype(jnp.int32) + (r_x < _NXM - 1).astype(jnp.int32)
           + (y > 0).astype(jnp.int32) + (y < _NYM - 1).astype(jnp.int32))
    rs = lax.rsqrt(deg.astype(f32))
    for g in range(_G):
        chans = [rs * o_ref[c * 242 + 2:c * 242 + 242, g * 66 + 1:g * 66 + 65]
                 + bg_ref[0:1, c:c + 1] for c in range(3)]
        blk = jnp.stack(chans, axis=-1).reshape(_NXM, _NYM * 3)
        out_ref[g * _NXM:(g + 1) * _NXM, :] = blk


def kernel(patch_vectors, W1, b1, W2, b2, Wg, bg, edge_index):
    del edge_index
    pv2 = patch_vectors.reshape(_NP, _IN)
    t_tab = pl.pallas_call(
        _field_body,
        out_shape=jax.ShapeDtypeStruct((_PROWS, _PCOLS), jnp.float32),
    )(pv2, W1, b1.reshape(1, _H1), W2, b2.reshape(1, _HID), Wg)

    mesh = plsc.VectorSubcoreMesh(core_axis_name="c", subcore_axis_name="s",
                                  num_cores=2, num_subcores=16)
    o_tab = pl.kernel(
        _sc_body,
        out_type=jax.ShapeDtypeStruct((_FLAT,), jnp.float32),
        mesh=mesh,
        scratch_types=[
            pltpu.VMEM((_WIN,), jnp.float32),
            pltpu.VMEM((_OUT,), jnp.float32),
        ],
    )(t_tab.reshape(_FLAT))

    out = pl.pallas_call(
        _final_body,
        out_shape=jax.ShapeDtypeStruct((_G * _NXM, _NYM * 3), jnp.float32),
    )(o_tab.reshape(_PROWS, _PCOLS), bg.reshape(1, 3))
    return out.reshape(_G, _NXM, _NYM, 3)


# speedup vs baseline: 2.7442x; 2.7442x over previous
import jax
import jax.numpy as jnp
from jax import lax
from jax.experimental import pallas as pl
from jax.experimental.pallas import tpu as pltpu
from jax.experimental.pallas import tpu_sc as plsc

_NXM, _NYM = 240, 64
_G = 8
_COLS = _G * _NYM
_NP = 480
_IN, _H1, _HID = 768, 512, 128

_PROWS, _PCOLS = 738, 528
_FLAT = _PROWS * _PCOLS
_TROWS = 23
_NTILES = 32
_WIN = (_TROWS + 2) * _PCOLS
_OUT = _TROWS * _PCOLS


def _field_body(pv_ref, w1_ref, b1_ref, w2_ref, b2_ref, wg_ref, t_ref):
    f32 = jnp.float32
    a = jnp.dot(pv_ref[...], w1_ref[...], preferred_element_type=f32)
    a = a + b1_ref[...]
    a = jnp.maximum(a, 0.0) + jnp.log1p(jnp.exp(-jnp.abs(a)))
    h = jnp.dot(a, w2_ref[...], preferred_element_type=f32) + b2_ref[...]
    p0 = jnp.dot(h, wg_ref[0:_HID, :], preferred_element_type=f32)

    r4 = lax.broadcasted_iota(jnp.int32, (_NP, _COLS), 0)
    c4 = lax.broadcasted_iota(jnp.int32, (_NP, _COLS), 1)
    mask = ((r4 // 60 == c4 // 64) & (r4 % 4 == (c4 % 64) // 16)).astype(f32)
    xu = lax.broadcasted_iota(jnp.int32, (_NXM, _NP), 0)
    ru = lax.broadcasted_iota(jnp.int32, (_NXM, _NP), 1)
    u = ((ru % 60) // 4 == xu // 16).astype(f32)

    r_x = lax.broadcasted_iota(jnp.int32, (_NXM, _COLS), 0)
    c_i = lax.broadcasted_iota(jnp.int32, (_NXM, _COLS), 1)
    y = c_i % _NYM
    deg = (1
           + (r_x > 0).astype(jnp.int32) + (r_x < _NXM - 1).astype(jnp.int32)
           + (y > 0).astype(jnp.int32) + (y < _NYM - 1).astype(jnp.int32))
    rs = lax.rsqrt(deg.astype(f32))
    xp_f = (r_x // 16).astype(f32)
    yp_f = (y // 16).astype(f32)
    xi_f = (r_x % 16).astype(f32) * (1.0 / 15.0)
    yi_f = (y % 16).astype(f32) * (1.0 / 15.0)

    t_ref[...] = jnp.zeros((_PROWS, _PCOLS), f32)
    for c in range(3):
        z = p0[:, c:c + 1] * mask
        b = jnp.dot(u, z, preferred_element_type=f32)
        hx = (b
              + xp_f * wg_ref[_HID + 0:_HID + 1, c:c + 1]
              + yp_f * wg_ref[_HID + 1:_HID + 2, c:c + 1]
              + xi_f * wg_ref[_HID + 2:_HID + 3, c:c + 1]
              + yi_f * wg_ref[_HID + 3:_HID + 4, c:c + 1])
        t = rs * hx
        for g in range(_G):
            t_ref[c * 242 + 2:c * 242 + 242, g * 66 + 1:g * 66 + 65] = (
                t[:, g * _NYM:(g + 1) * _NYM])


def _sc_body(t_hbm, o_hbm, win, obuf):
    wid = lax.axis_index("s") * 2 + lax.axis_index("c")
    base = wid * _OUT
    pltpu.sync_copy(t_hbm.at[pl.ds(base, _WIN)], win)

    def body(i, carry):
        off = i * 16
        mid = win[pl.ds(off + _PCOLS, 16)]
        up = win[pl.ds(off, 16)]
        dn = win[pl.ds(off + 2 * _PCOLS, 16)]
        lf = win[pl.ds(off + _PCOLS - 1, 16)]
        rt = win[pl.ds(off + _PCOLS + 1, 16)]
        obuf[pl.ds(off, 16)] = mid + up + dn + lf + rt
        return carry

    lax.fori_loop(0, _OUT // 16, body, 0)
    pltpu.sync_copy(obuf, o_hbm.at[pl.ds(base + _PCOLS, _OUT)])


def _final_body(o_ref, bg_ref, out_ref):
    f32 = jnp.float32
    r_x = lax.broadcasted_iota(jnp.int32, (_NXM, _COLS), 0)
    c_i = lax.broadcasted_iota(jnp.int32, (_NXM, _COLS), 1)
    y = c_i % _NYM
    deg = (1
           + (r_x > 0).astype(jnp.int32) + (r_x < _NXM - 1).astype(jnp.int32)
           + (y > 0).astype(jnp.int32) + (y < _NYM - 1).astype(jnp.int32))
    rs = lax.rsqrt(deg.astype(f32))
    for c in range(3):
        plane = jnp.concatenate(
            [o_ref[c * 242 + 2:c * 242 + 242, g * 66 + 1:g * 66 + 65]
             for g in range(_G)], axis=1)
        out_ref[c, :, :] = rs * plane + bg_ref[0:1, c:c + 1]


def kernel(patch_vectors, W1, b1, W2, b2, Wg, bg, edge_index):
    del edge_index
    pv2 = patch_vectors.reshape(_NP, _IN)
    t_tab = pl.pallas_call(
        _field_body,
        out_shape=jax.ShapeDtypeStruct((_PROWS, _PCOLS), jnp.float32),
    )(pv2, W1, b1.reshape(1, _H1), W2, b2.reshape(1, _HID), Wg)

    mesh = plsc.VectorSubcoreMesh(core_axis_name="c", subcore_axis_name="s",
                                  num_cores=2, num_subcores=16)
    o_tab = pl.kernel(
        _sc_body,
        out_type=jax.ShapeDtypeStruct((_FLAT,), jnp.float32),
        mesh=mesh,
        scratch_types=[
            pltpu.VMEM((_WIN,), jnp.float32),
            pltpu.VMEM((_OUT,), jnp.float32),
        ],
    )(t_tab.reshape(_FLAT))

    out = pl.pallas_call(
        _final_body,
        out_shape=jax.ShapeDtypeStruct((3, _NXM, _COLS), jnp.float32),
    )(o_tab.reshape(_PROWS, _PCOLS), bg.reshape(1, 3))
    return out.reshape(3, _NXM, _G, _NYM).transpose(2, 1, 3, 0)
